# baseline (speedup 1.0000x reference)
import functools

import jax
import jax.numpy as jnp
from jax import lax
from jax.experimental import pallas as pl
from jax.experimental.pallas import tpu as pltpu
from jax.experimental.pallas import tpu_sc as plsc

END_ID = 2
B = 128
V = 100000
T = 2048
NC = 2
NS = 16
L = 16
NW = NC * NS
RPW = B // NW
NVEC = V // L
CH_SIZES = (19968, 19968, 19968, 19968, 20128)
CH_OFFS = (0, 19968, 39936, 59904, 79872)
NCH = len(CH_SIZES)
CH_MAX = max(CH_SIZES)
NACC = 4

_mesh = plsc.VectorSubcoreMesh(core_axis_name="c", subcore_axis_name="s")


@functools.partial(
    pl.kernel,
    out_type=[
        jax.ShapeDtypeStruct((B, T), jnp.int32),
        jax.ShapeDtypeStruct((NW, L), jnp.int32),
    ],
    mesh=_mesh,
    compiler_params=pltpu.CompilerParams(needs_layout_passes=False),
    scratch_types=[
        pltpu.VMEM((CH_MAX,), jnp.float32),
        pltpu.VMEM((CH_MAX,), jnp.float32),
        pltpu.VMEM((RPW, T), jnp.int32),
        pltpu.VMEM((8,), jnp.int32),
        pltpu.VMEM((L,), jnp.int32),
        pltpu.VMEM((L,), jnp.int32),
        pltpu.SemaphoreType.DMA,
        pltpu.SemaphoreType.DMA,
    ],
)
def _sc_greedy(hid, upd16, outin, flags8, out, flstage,
               buf0, buf1, outbuf, fbuf, ubuf, vbuf, sem0, sem1):
    wid = lax.axis_index("s") * NC + lax.axis_index("c")
    base = wid * RPW
    lanes = lax.iota(jnp.int32, L)
    bufs, sems = (buf0, buf1), (sem0, sem1)

    pltpu.sync_copy(flags8.at[wid], fbuf)
    pltpu.sync_copy(upd16, ubuf)

    seq = [(r, c) for r in range(RPW) for c in range(NCH)]

    def start(k):
        r, c = seq[k]
        return pltpu.async_copy(
            hid.at[base + r, 0, pl.ds(CH_OFFS[c], CH_SIZES[c])],
            bufs[k % 2].at[pl.ds(0, CH_SIZES[c])], sems[k % 2])

    def fresh_accs():
        return (tuple(jnp.full((L,), -jnp.inf, jnp.float32) for _ in range(NACC)),
                tuple(jnp.zeros((L,), jnp.int32) for _ in range(NACC)))

    def step_one(buf, vec_i, gvec_i, vmax, vj):
        v = buf[pl.ds(vec_i * L, L)]
        msk = v > vmax
        return (jnp.where(msk, v, vmax),
                jnp.where(msk, jnp.full((L,), gvec_i, jnp.int32), vj))

    handle = start(0)
    accs = fresh_accs()
    winners = jnp.zeros((L,), jnp.int32)
    for k, (r, c) in enumerate(seq):
        nxt = start(k + 1) if k + 1 < len(seq) else None
        handle.wait()
        handle = nxt
        buf = bufs[k % 2]
        gbase = CH_OFFS[c] // L
        ch_v = CH_SIZES[c] // L
        main_v = (ch_v // NACC) * NACC

        def body(i, carry):
            vmaxs, vjs = carry
            nvm, nvj = [], []
            for a in range(NACC):
                vm, vj = step_one(buf, i + a, i + (gbase + a),
                                  vmaxs[a], vjs[a])
                nvm.append(vm)
                nvj.append(vj)
            return tuple(nvm), tuple(nvj)

        accs = plsc.parallel_loop(0, main_v, NACC, unroll=2, carry=accs)(body)

        vmaxs, vjs = (list(accs[0]), list(accs[1]))
        for t in range(main_v, ch_v):
            a = t - main_v
            vmaxs[a], vjs[a] = step_one(buf, t, gbase + t, vmaxs[a], vjs[a])
        accs = (tuple(vmaxs), tuple(vjs))

        if c == NCH - 1:
            vmaxs, vjs = accs
            m = vmaxs[0]
            g = vjs[0] * L + lanes
            for a in range(1, NACC):
                g2 = vjs[a] * L + lanes
                better = (vmaxs[a] > m) | ((vmaxs[a] == m) & (g2 < g))
                m = jnp.where(better, vmaxs[a], m)
                g = jnp.where(better, g2, g)
            for kk in (8, 4, 2, 1):
                idx = lanes ^ kk
                m2 = m.at[idx].get(mode="promise_in_bounds")
                g2 = g.at[idx].get(mode="promise_in_bounds")
                better = (m2 > m) | ((m2 == m) & (g2 < g))
                m = jnp.where(better, m2, m)
                g = jnp.where(better, g2, g)
            winners = jnp.where(lanes == r, g, winners)
            accs = fresh_accs()

    fl = plsc.load_gather(fbuf, [lanes & 3])
    y = jnp.where(fl != 0, winners, jnp.full((L,), END_ID, jnp.int32))
    flnew = (y != END_ID).astype(jnp.int32)

    pltpu.sync_copy(outin.at[pl.ds(base, RPW)], outbuf)
    uvec = ubuf[...]
    plsc.store_scatter(outbuf, [lanes, uvec], y, mask=lanes < RPW)
    pltpu.sync_copy(outbuf, out.at[pl.ds(base, RPW)])

    vbuf[...] = flnew
    pltpu.sync_copy(vbuf, flstage.at[wid])


def kernel(hidden_state, update_index, out_ids, flags):
    upd16 = jnp.full((L,), update_index, jnp.int32)
    flags8 = jnp.zeros((NW, 8), jnp.int32).at[:, :RPW].set(
        flags.reshape(NW, RPW).astype(jnp.int32))
    out, flstage = _sc_greedy(hidden_state, upd16, out_ids, flags8)
    flags_new = flstage[:, :RPW].reshape(B, 1).astype(jnp.bool_)
    return out, flags_new


# speedup vs baseline: 2.9613x; 2.9613x over previous
import functools

import jax
import jax.numpy as jnp
from jax import lax
from jax.experimental import pallas as pl
from jax.experimental.pallas import tpu as pltpu
from jax.experimental.pallas import tpu_sc as plsc

END_ID = 2
B = 128
V = 100000
T = 2048
NC = 2
NS = 16
L = 16
NW = NC * NS
BG = B // L
SPAN = 3128
V0_LAST = V - SPAN
CHUNKS = (400, 400, 400, 400, 400, 400, 400, 328)
CH_MAX = max(CHUNKS)

_mesh = plsc.VectorSubcoreMesh(core_axis_name="c", subcore_axis_name="s")


@functools.partial(
    pl.kernel,
    out_type=[
        jax.ShapeDtypeStruct((NW, B), jnp.float32),
        jax.ShapeDtypeStruct((NW, B), jnp.int32),
    ],
    mesh=_mesh,
    compiler_params=pltpu.CompilerParams(needs_layout_passes=False),
    scratch_types=[
        pltpu.VMEM((CH_MAX, B), jnp.float32),
        pltpu.VMEM((CH_MAX, B), jnp.float32),
        pltpu.VMEM((B,), jnp.float32),
        pltpu.VMEM((B,), jnp.int32),
        pltpu.SemaphoreType.DMA,
        pltpu.SemaphoreType.DMA,
    ],
)
def _sc_scan(hid, valstage, idxstage, buf0, buf1, vrow, irow, sem0, sem1):
    wid = lax.axis_index("s") * NC + lax.axis_index("c")
    bufs, sems = (buf0, buf1), (sem0, sem1)
    v0 = pl.multiple_of(jnp.minimum(wid * SPAN, V0_LAST), 8)

    offs = [0]
    for c in CHUNKS:
        offs.append(offs[-1] + c)

    def start(k):
        n = CHUNKS[k]
        return pltpu.async_copy(
            hid.at[pl.ds(v0 + offs[k], n)],
            bufs[k % 2].at[pl.ds(0, n)], sems[k % 2])

    vmaxs = [jnp.full((L,), -jnp.inf, jnp.float32) for _ in range(BG)]
    vjs = [jnp.zeros((L,), jnp.int32) for _ in range(BG)]

    handle = start(0)
    for k, n in enumerate(CHUNKS):
        nxt = start(k + 1) if k + 1 < len(CHUNKS) else None
        handle.wait()
        handle = nxt
        buf = bufs[k % 2]
        vbase = v0 + offs[k]

        def body(i, carry):
            vm, vj = carry
            jvec = jnp.full((L,), vbase + i, jnp.int32)
            nvm, nvj = [], []
            for g in range(BG):
                v = buf[i, pl.ds(g * L, L)]
                msk = v > vm[g]
                nvm.append(jnp.where(msk, v, vm[g]))
                nvj.append(jnp.where(msk, jvec, vj[g]))
            return tuple(nvm), tuple(nvj)

        vmaxs, vjs = plsc.parallel_loop(
            0, n, 1, unroll=2, carry=(tuple(vmaxs), tuple(vjs)))(body)
        vmaxs, vjs = list(vmaxs), list(vjs)

    for g in range(BG):
        vrow[pl.ds(g * L, L)] = vmaxs[g]
        irow[pl.ds(g * L, L)] = vjs[g]
    pltpu.sync_copy(vrow, valstage.at[wid])
    pltpu.sync_copy(irow, idxstage.at[wid])


@functools.partial(
    pl.kernel,
    out_type=[
        jax.ShapeDtypeStruct((B, T), jnp.int32),
        jax.ShapeDtypeStruct((NW, L), jnp.int32),
    ],
    mesh=_mesh,
    compiler_params=pltpu.CompilerParams(needs_layout_passes=False),
    scratch_types=[
        pltpu.VMEM((NW, B), jnp.float32),
        pltpu.VMEM((NW, B), jnp.int32),
        pltpu.VMEM((4, T), jnp.int32),
        pltpu.VMEM((L,), jnp.int32),
        pltpu.VMEM((L,), jnp.int32),
        pltpu.VMEM((L,), jnp.int32),
    ],
)
def _sc_merge(valstage, idxstage, upd16, outin, flags128, out, flstage,
              valb, idxb, outbuf, fbuf, ubuf, vbuf):
    wid = lax.axis_index("s") * NC + lax.axis_index("c")
    bg = wid // 4
    lanes = lax.iota(jnp.int32, L)

    pltpu.sync_copy(valstage, valb)
    pltpu.sync_copy(idxstage, idxb)
    pltpu.sync_copy(flags128.at[pl.ds(pl.multiple_of(bg * L, 8), L)], fbuf)
    pltpu.sync_copy(upd16, ubuf)

    goff = bg * L
    m = valb[0, pl.ds(goff, L)]
    g = idxb[0, pl.ds(goff, L)]
    for w1 in range(1, NW):
        m2 = valb[w1, pl.ds(goff, L)]
        g2 = idxb[w1, pl.ds(goff, L)]
        better = (m2 > m) | ((m2 == m) & (g2 < g))
        m = jnp.where(better, m2, m)
        g = jnp.where(better, g2, g)

    fl = fbuf[...]
    y16 = jnp.where(fl != 0, g, jnp.full((L,), END_ID, jnp.int32))
    flnew = (y16 != END_ID).astype(jnp.int32)

    y4 = y16.at[(wid % 4) * 4 + (lanes & 3)].get(mode="promise_in_bounds")
    pltpu.sync_copy(outin.at[pl.ds(wid * 4, 4)], outbuf)
    plsc.store_scatter(outbuf, [lanes, ubuf[...]], y4, mask=lanes < 4)
    pltpu.sync_copy(outbuf, out.at[pl.ds(wid * 4, 4)])

    vbuf[...] = flnew
    pltpu.sync_copy(vbuf, flstage.at[wid])


def kernel(hidden_state, update_index, out_ids, flags):
    hid = jnp.transpose(hidden_state, (1, 2, 0)).reshape(V, B)
    upd16 = jnp.full((L,), update_index, jnp.int32)
    flags128 = flags.reshape(B).astype(jnp.int32)
    valstage, idxstage = _sc_scan(hid)
    out, flstage = _sc_merge(valstage, idxstage, upd16, out_ids, flags128)
    flags_new = flstage[0::4].reshape(B, 1).astype(jnp.bool_)
    return out, flags_new
